# baseline (speedup 1.0000x reference)
import functools

import jax
import jax.numpy as jnp
from jax.experimental import pallas as pl
from jax.experimental.pallas import tpu as pltpu


def _linear_kernel(x_ref, w_ref, b_ref, h_ref):
    h_ref[...] = (
        jnp.dot(x_ref[...], w_ref[...].T, preferred_element_type=jnp.float32)
        + b_ref[...]
    )


def _matmul_relu_kernel(adj_ref, h_ref, out_ref):
    out_ref[...] = jnp.maximum(
        jnp.dot(adj_ref[...], h_ref[...], preferred_element_type=jnp.float32),
        0.0,
    )


@jax.jit
def kernel(x, adj, W, b):
    n, d_in = x.shape
    d_out = W.shape[0]
    b2 = b.reshape(1, d_out)

    bm_lin = 1000
    h = pl.pallas_call(
        _linear_kernel,
        grid=(n // bm_lin,),
        in_specs=[
            pl.BlockSpec((bm_lin, d_in), lambda i: (i, 0)),
            pl.BlockSpec((d_in, d_out), lambda i: (0, 0)),
            pl.BlockSpec((1, d_out), lambda i: (0, 0)),
        ],
        out_specs=pl.BlockSpec((bm_lin, d_out), lambda i: (i, 0)),
        out_shape=jax.ShapeDtypeStruct((n, d_out), jnp.float32),
    )(x, W, b2)

    bm = 400
    out = pl.pallas_call(
        _matmul_relu_kernel,
        grid=(n // bm,),
        in_specs=[
            pl.BlockSpec((bm, n), lambda m: (m, 0)),
            pl.BlockSpec((n, d_out), lambda m: (0, 0)),
        ],
        out_specs=pl.BlockSpec((bm, d_out), lambda m: (m, 0)),
        out_shape=jax.ShapeDtypeStruct((n, d_out), jnp.float32),
        compiler_params=pltpu.CompilerParams(
            dimension_semantics=("parallel",),
        ),
    )(adj, h)
    return out


# speedup vs baseline: 1.0798x; 1.0798x over previous
import jax
import jax.numpy as jnp
from jax.experimental import pallas as pl
from jax.experimental.pallas import tpu as pltpu


def _gcn_kernel(x_ref, w_ref, b_ref, adj_ref, out_ref, h_ref):
    @pl.when(pl.program_id(0) == 0)
    def _():
        h_ref[...] = (
            jnp.dot(x_ref[...], w_ref[...].T, preferred_element_type=jnp.float32)
            + b_ref[...]
        )

    out_ref[...] = jnp.maximum(
        jnp.dot(adj_ref[...], h_ref[...], preferred_element_type=jnp.float32),
        0.0,
    )


@jax.jit
def kernel(x, adj, W, b):
    n, d_in = x.shape
    d_out = W.shape[0]
    b2 = b.reshape(1, d_out)

    bm = 400
    out = pl.pallas_call(
        _gcn_kernel,
        grid=(n // bm,),
        in_specs=[
            pl.BlockSpec((n, d_in), lambda m: (0, 0)),
            pl.BlockSpec((d_in, d_out), lambda m: (0, 0)),
            pl.BlockSpec((1, d_out), lambda m: (0, 0)),
            pl.BlockSpec((bm, n), lambda m: (m, 0)),
        ],
        out_specs=pl.BlockSpec((bm, d_out), lambda m: (m, 0)),
        out_shape=jax.ShapeDtypeStruct((n, d_out), jnp.float32),
        scratch_shapes=[pltpu.VMEM((n, d_out), jnp.float32)],
        compiler_params=pltpu.CompilerParams(
            dimension_semantics=("arbitrary",),
        ),
    )(x, W, b2, adj)
    return out
